# baseline (speedup 1.0000x reference)
import functools

import jax
import jax.numpy as jnp
from jax import lax
from jax.experimental import pallas as pl
from jax.experimental.pallas import tpu as pltpu
from jax.experimental.pallas import tpu_sc as plsc

DIM = 64
EPS = 1e-5
NC = 2
NS = 16
NW = NC * NS
CHUNK = 128
NBUF = 5


@functools.lru_cache(maxsize=None)
def _build_sc_gather(n_tokens, vocab):
    tok_per_w = n_tokens // NW
    rounds = tok_per_w // CHUNK
    assert rounds % NBUF == 0

    mesh = plsc.VectorSubcoreMesh(
        core_axis_name="c", subcore_axis_name="s", num_cores=NC, num_subcores=NS
    )

    @functools.partial(
        pl.kernel,
        out_type=jax.ShapeDtypeStruct((n_tokens, 2 * DIM), jnp.float32),
        mesh=mesh,
        scratch_types=[
            pltpu.VMEM((rounds, CHUNK), jnp.int32),
            pltpu.VMEM((NBUF, CHUNK, DIM), jnp.float32),
            [pltpu.SemaphoreType.DMA] * NBUF,
            [pltpu.SemaphoreType.DMA] * NBUF,
        ],
        compiler_params=pltpu.CompilerParams(use_tc_tiling_on_sc=False),
    )
    def sc_gather(ids_hbm, tok_hbm, out_hbm, ids_v, rows_v, gsems, ssems):
        wid = lax.axis_index("s") * NC + lax.axis_index("c")
        base = wid * tok_per_w

        pltpu.sync_copy(ids_hbm.at[wid], ids_v)

        def start_gather(g, b):
            pltpu.async_copy(tok_hbm.at[ids_v.at[g]], rows_v.at[b], gsems[b])

        def wait_gather(g, b):
            pltpu.make_async_copy(
                tok_hbm.at[ids_v.at[g]], rows_v.at[b], gsems[b]).wait()

        def out_slice(g):
            return out_hbm.at[pl.ds(base + g * CHUNK, CHUNK), pl.ds(0, DIM)]

        def start_store(g, b):
            pltpu.async_copy(rows_v.at[b], out_slice(g), ssems[b])

        def wait_store(g, b):
            pltpu.make_async_copy(rows_v.at[b], out_slice(g), ssems[b]).wait()

        for b in range(NBUF):
            start_gather(b, b)

        def ring(r, carry):
            for b in range(NBUF):
                g = NBUF * r + b
                wait_gather(g, b)
                start_store(g, b)

                @pl.when(g + NBUF < rounds)
                def _():
                    wait_store(g, b)
                    start_gather(g + NBUF, b)
            return carry

        lax.fori_loop(0, rounds // NBUF, ring, 0)
        for b in range(NBUF):
            wait_store(rounds - NBUF + b, b)

    return sc_gather


@functools.lru_cache(maxsize=None)
def _build_tc_ln(bsz, seq, blk_b):
    def body(g_ref, ty_ref, pos_ref, seg_ref, gam_ref, bet_ref, o_ref):
        pos = pos_ref[...]
        seg0 = seg_ref[0]
        segd1 = (seg_ref[1] - seg0).reshape(1, DIM)
        gam = gam_ref[...]
        bet = bet_ref[...]
        for i in range(blk_b):
            x = g_ref[pl.ds(i * seq, seq), pl.ds(0, DIM)]
            tyrow = ty_ref[pl.ds(i, 1), :].astype(jnp.float32)
            segc = lax.dot_general(
                tyrow, segd1, (((0,), (0,)), ((), ())),
                precision=lax.Precision.HIGHEST,
                preferred_element_type=jnp.float32)
            xi = x + pos + seg0 + segc
            mean = jnp.mean(xi, axis=-1, keepdims=True)
            xc = xi - mean
            var = jnp.mean(xc * xc, axis=-1, keepdims=True)
            y = xc * lax.rsqrt(var + EPS)
            o_ref[i] = y * gam + bet

    grid = bsz // blk_b
    return pl.pallas_call(
        body,
        grid=(grid,),
        in_specs=[
            pl.BlockSpec((blk_b * seq, 2 * DIM), lambda g: (g, 0)),
            pl.BlockSpec((blk_b, seq), lambda g: (g, 0)),
            pl.BlockSpec((seq, DIM), lambda g: (0, 0)),
            pl.BlockSpec((2, DIM), lambda g: (0, 0)),
            pl.BlockSpec((DIM,), lambda g: (0,)),
            pl.BlockSpec((DIM,), lambda g: (0,)),
        ],
        out_specs=pl.BlockSpec((blk_b, seq, DIM), lambda g: (g, 0, 0)),
        out_shape=jax.ShapeDtypeStruct((bsz, seq, DIM), jnp.float32),
    )


def kernel(input_ids, token_type_ids, token_table, segment_table,
           position_table, ln_gamma, ln_beta):
    bsz, seq = input_ids.shape
    n_tokens = bsz * seq
    vocab, dim = token_table.shape
    assert dim == DIM and n_tokens % (NW * CHUNK) == 0 and seq % 2 == 0

    tok_per_w = n_tokens // NW
    rounds = tok_per_w // CHUNK
    ids = input_ids.reshape(NW, rounds, CHUNK).astype(jnp.int32)

    gath = _build_sc_gather(n_tokens, vocab)(ids, token_table)
    tys = token_type_ids.astype(jnp.int32)

    blk_b = 16
    out = _build_tc_ln(bsz, seq, blk_b)(
        gath, tys, position_table, segment_table, ln_gamma, ln_beta)
    return out


# speedup vs baseline: 1.0018x; 1.0018x over previous
import functools

import jax
import jax.numpy as jnp
from jax import lax
from jax.experimental import pallas as pl
from jax.experimental.pallas import tpu as pltpu
from jax.experimental.pallas import tpu_sc as plsc

DIM = 64
EPS = 1e-5
NC = 2
NS = 16
NW = NC * NS
CHUNK = 128
NBUF = 5


@functools.lru_cache(maxsize=None)
def _build_sc_gather(n_tokens, vocab):
    tok_per_w = n_tokens // NW
    rounds = tok_per_w // CHUNK
    assert rounds % NBUF == 0

    mesh = plsc.VectorSubcoreMesh(
        core_axis_name="c", subcore_axis_name="s", num_cores=NC, num_subcores=NS
    )

    @functools.partial(
        pl.kernel,
        out_type=jax.ShapeDtypeStruct((n_tokens, 2 * DIM), jnp.float32),
        mesh=mesh,
        scratch_types=[
            pltpu.VMEM((rounds, CHUNK), jnp.int32),
            pltpu.VMEM((NBUF, CHUNK, DIM), jnp.float32),
            [pltpu.SemaphoreType.DMA] * NBUF,
            [pltpu.SemaphoreType.DMA] * NBUF,
        ],
        compiler_params=pltpu.CompilerParams(use_tc_tiling_on_sc=False),
    )
    def sc_gather(ids_hbm, tok_hbm, out_hbm, ids_v, rows_v, gsems, ssems):
        wid = lax.axis_index("s") * NC + lax.axis_index("c")
        base = wid * tok_per_w

        pltpu.sync_copy(ids_hbm.at[wid], ids_v)

        def start_gather(g, b):
            pltpu.async_copy(tok_hbm.at[ids_v.at[g]], rows_v.at[b], gsems[b])

        def wait_gather(g, b):
            pltpu.make_async_copy(
                tok_hbm.at[ids_v.at[g]], rows_v.at[b], gsems[b]).wait()

        def out_slice(g):
            return out_hbm.at[pl.ds(base + g * CHUNK, CHUNK), pl.ds(0, DIM)]

        def start_store(g, b):
            pltpu.async_copy(rows_v.at[b], out_slice(g), ssems[b])

        def wait_store(g, b):
            pltpu.make_async_copy(rows_v.at[b], out_slice(g), ssems[b]).wait()

        for b in range(NBUF):
            start_gather(b, b)

        def ring(r, carry):
            for b in range(NBUF):
                g = NBUF * r + b
                wait_gather(g, b)
                pltpu.sync_copy(rows_v.at[b], out_slice(g))

                @pl.when(g + NBUF < rounds)
                def _():
                    start_gather(g + NBUF, b)
            return carry

        lax.fori_loop(0, rounds // NBUF, ring, 0)

    return sc_gather


@functools.lru_cache(maxsize=None)
def _build_tc_ln(bsz, seq, blk_b):
    def body(g_ref, ty_ref, pos_ref, seg_ref, gam_ref, bet_ref, o_ref):
        pos = pos_ref[...]
        seg0 = seg_ref[0]
        segd1 = (seg_ref[1] - seg0).reshape(1, DIM)
        gam = gam_ref[...]
        bet = bet_ref[...]
        for i in range(blk_b):
            x = g_ref[pl.ds(i * seq, seq), pl.ds(0, DIM)]
            tyrow = ty_ref[pl.ds(i, 1), :].astype(jnp.float32)
            segc = lax.dot_general(
                tyrow, segd1, (((0,), (0,)), ((), ())),
                precision=lax.Precision.HIGHEST,
                preferred_element_type=jnp.float32)
            xi = x + pos + seg0 + segc
            mean = jnp.mean(xi, axis=-1, keepdims=True)
            xc = xi - mean
            var = jnp.mean(xc * xc, axis=-1, keepdims=True)
            y = xc * lax.rsqrt(var + EPS)
            o_ref[i] = y * gam + bet

    grid = bsz // blk_b
    return pl.pallas_call(
        body,
        grid=(grid,),
        in_specs=[
            pl.BlockSpec((blk_b * seq, 2 * DIM), lambda g: (g, 0)),
            pl.BlockSpec((blk_b, seq), lambda g: (g, 0)),
            pl.BlockSpec((seq, DIM), lambda g: (0, 0)),
            pl.BlockSpec((2, DIM), lambda g: (0, 0)),
            pl.BlockSpec((DIM,), lambda g: (0,)),
            pl.BlockSpec((DIM,), lambda g: (0,)),
        ],
        out_specs=pl.BlockSpec((blk_b, seq, DIM), lambda g: (g, 0, 0)),
        out_shape=jax.ShapeDtypeStruct((bsz, seq, DIM), jnp.float32),
    )


def kernel(input_ids, token_type_ids, token_table, segment_table,
           position_table, ln_gamma, ln_beta):
    bsz, seq = input_ids.shape
    n_tokens = bsz * seq
    vocab, dim = token_table.shape
    assert dim == DIM and n_tokens % (NW * CHUNK) == 0 and seq % 2 == 0

    tok_per_w = n_tokens // NW
    rounds = tok_per_w // CHUNK
    ids = input_ids.reshape(NW, rounds, CHUNK).astype(jnp.int32)

    gath = _build_sc_gather(n_tokens, vocab)(ids, token_table)
    tys = token_type_ids.astype(jnp.int32)

    blk_b = 16
    out = _build_tc_ln(bsz, seq, blk_b)(
        gath, tys, position_table, segment_table, ln_gamma, ln_beta)
    return out


# speedup vs baseline: 1.1637x; 1.1617x over previous
import functools

import jax
import jax.numpy as jnp
from jax import lax
from jax.experimental import pallas as pl
from jax.experimental.pallas import tpu as pltpu
from jax.experimental.pallas import tpu_sc as plsc

DIM = 64
EPS = 1e-5
NC = 2
NS = 16
NW = NC * NS
CHUNK = 128
NBUF = 5


@functools.lru_cache(maxsize=None)
def _build_sc_gather(n_tokens, vocab):
    tok_per_w = n_tokens // NW
    rounds = tok_per_w // CHUNK
    assert rounds % NBUF == 0

    mesh = plsc.VectorSubcoreMesh(
        core_axis_name="c", subcore_axis_name="s", num_cores=NC, num_subcores=NS
    )

    @functools.partial(
        pl.kernel,
        out_type=jax.ShapeDtypeStruct((n_tokens, 2 * DIM), jnp.float32),
        mesh=mesh,
        scratch_types=[
            pltpu.VMEM((rounds, CHUNK), jnp.int32),
            pltpu.VMEM((NBUF, CHUNK, DIM), jnp.float32),
            [pltpu.SemaphoreType.DMA] * NBUF,
            [pltpu.SemaphoreType.DMA] * NBUF,
        ],
        compiler_params=pltpu.CompilerParams(use_tc_tiling_on_sc=False),
    )
    def sc_gather(ids_hbm, tok_hbm, out_hbm, ids_v, rows_v, gsems, ssems):
        wid = lax.axis_index("s") * NC + lax.axis_index("c")
        base = wid * tok_per_w

        pltpu.sync_copy(ids_hbm.at[wid], ids_v)

        def start_gather(g, b):
            pltpu.async_copy(tok_hbm.at[ids_v.at[g]], rows_v.at[b], gsems[b])

        def wait_gather(g, b):
            pltpu.make_async_copy(
                tok_hbm.at[ids_v.at[g]], rows_v.at[b], gsems[b]).wait()

        def out_slice(g):
            return out_hbm.at[pl.ds(base + g * CHUNK, CHUNK), pl.ds(0, DIM)]

        def start_store(g, b):
            pltpu.async_copy(rows_v.at[b], out_slice(g), ssems[b])

        def wait_store(g, b):
            pltpu.make_async_copy(rows_v.at[b], out_slice(g), ssems[b]).wait()

        for b in range(NBUF):
            start_gather(b, b)

        def ring(r, carry):
            for b in range(NBUF):
                g = NBUF * r + b
                wait_gather(g, b)
                pltpu.sync_copy(rows_v.at[b], out_slice(g))

                @pl.when(g + NBUF < rounds)
                def _():
                    start_gather(g + NBUF, b)
            return carry

        lax.fori_loop(0, rounds // NBUF, ring, 0)

    return sc_gather


@functools.lru_cache(maxsize=None)
def _build_tc_ln(bsz, seq, blk_b):
    def body(g_ref, ty_ref, pos_ref, seg_ref, gam_ref, bet_ref, o_ref):
        pos_t = pos_ref[...]
        seg0 = seg_ref[:, 0:1]
        segd = seg_ref[:, 1:2] - seg0
        gam = gam_ref[...]
        bet = bet_ref[...]
        for i in range(blk_b):
            xt = jnp.transpose(
                g_ref[pl.ds(i * seq, seq), pl.ds(0, DIM)])
            tyr = ty_ref[pl.ds(i, 1), :].astype(jnp.float32)
            x = xt + pos_t + seg0 + tyr * segd
            mean = jnp.mean(x, axis=0, keepdims=True)
            xc = x - mean
            var = jnp.mean(xc * xc, axis=0, keepdims=True)
            y = xc * lax.rsqrt(var + EPS)
            o_ref[i] = y * gam + bet

    grid = bsz // blk_b
    return pl.pallas_call(
        body,
        grid=(grid,),
        in_specs=[
            pl.BlockSpec((blk_b * seq, 2 * DIM), lambda g: (g, 0)),
            pl.BlockSpec((blk_b, seq), lambda g: (g, 0)),
            pl.BlockSpec((DIM, seq), lambda g: (0, 0)),
            pl.BlockSpec((DIM, 2), lambda g: (0, 0)),
            pl.BlockSpec((DIM, 1), lambda g: (0, 0)),
            pl.BlockSpec((DIM, 1), lambda g: (0, 0)),
        ],
        out_specs=pl.BlockSpec((blk_b, DIM, seq), lambda g: (g, 0, 0)),
        out_shape=jax.ShapeDtypeStruct((bsz, DIM, seq), jnp.float32),
    )


def kernel(input_ids, token_type_ids, token_table, segment_table,
           position_table, ln_gamma, ln_beta):
    bsz, seq = input_ids.shape
    n_tokens = bsz * seq
    vocab, dim = token_table.shape
    assert dim == DIM and n_tokens % (NW * CHUNK) == 0 and seq % 2 == 0

    tok_per_w = n_tokens // NW
    rounds = tok_per_w // CHUNK
    ids = input_ids.reshape(NW, rounds, CHUNK).astype(jnp.int32)

    gath = _build_sc_gather(n_tokens, vocab)(ids, token_table)
    tys = token_type_ids.astype(jnp.int32)

    blk_b = 16
    out_t = _build_tc_ln(bsz, seq, blk_b)(
        gath, tys, position_table.T, segment_table.T,
        ln_gamma[:, None], ln_beta[:, None])
    return out_t.transpose(0, 2, 1)


# speedup vs baseline: 1.2064x; 1.0367x over previous
import functools

import jax
import jax.numpy as jnp
from jax import lax
from jax.experimental import pallas as pl
from jax.experimental.pallas import tpu as pltpu
from jax.experimental.pallas import tpu_sc as plsc

DIM = 64
EPS = 1e-5
NC = 2
NS = 16
NW = NC * NS
CHUNK = 128
NBUF = 5


@functools.lru_cache(maxsize=None)
def _build_sc_gather(n_tokens, vocab):
    tok_per_w = n_tokens // NW
    rounds = tok_per_w // CHUNK
    assert rounds % NBUF == 0

    mesh = plsc.VectorSubcoreMesh(
        core_axis_name="c", subcore_axis_name="s", num_cores=NC, num_subcores=NS
    )

    @functools.partial(
        pl.kernel,
        out_type=jax.ShapeDtypeStruct((n_tokens, 2 * DIM), jnp.float32),
        mesh=mesh,
        scratch_types=[
            pltpu.VMEM((rounds, CHUNK), jnp.int32),
            pltpu.VMEM((NBUF, CHUNK, DIM), jnp.float32),
            [pltpu.SemaphoreType.DMA] * NBUF,
            [pltpu.SemaphoreType.DMA] * NBUF,
        ],
        compiler_params=pltpu.CompilerParams(use_tc_tiling_on_sc=False),
    )
    def sc_gather(ids_hbm, tok_hbm, out_hbm, ids_v, rows_v, gsems, ssems):
        wid = lax.axis_index("s") * NC + lax.axis_index("c")
        base = wid * tok_per_w

        pltpu.sync_copy(ids_hbm.at[wid], ids_v)

        def start_gather(g, b):
            pltpu.async_copy(tok_hbm.at[ids_v.at[g]], rows_v.at[b], gsems[b])

        def wait_gather(g, b):
            pltpu.make_async_copy(
                tok_hbm.at[ids_v.at[g]], rows_v.at[b], gsems[b]).wait()

        def out_slice(g):
            return out_hbm.at[pl.ds(base + g * CHUNK, CHUNK), pl.ds(0, DIM)]

        def start_store(g, b):
            pltpu.async_copy(rows_v.at[b], out_slice(g), ssems[b])

        def wait_store(g, b):
            pltpu.make_async_copy(rows_v.at[b], out_slice(g), ssems[b]).wait()

        for b in range(NBUF):
            start_gather(b, b)

        def ring(r, carry):
            for b in range(NBUF):
                g = NBUF * r + b
                wait_gather(g, b)
                pltpu.sync_copy(rows_v.at[b], out_slice(g))

                @pl.when(g + NBUF < rounds)
                def _():
                    start_gather(g + NBUF, b)
            return carry

        lax.fori_loop(0, rounds // NBUF, ring, 0)

    return sc_gather


@functools.lru_cache(maxsize=None)
def _build_tc_ln(bsz, seq, blk_b):
    def body(g_ref, ty_ref, pos_ref, seg_ref, gam_ref, bet_ref, o_ref):
        pos_t = pos_ref[...]
        seg0 = seg_ref[:, 0:1]
        segd = seg_ref[:, 1:2] - seg0
        gam = gam_ref[...]
        bet = bet_ref[...]
        for i in range(blk_b):
            xt = jnp.transpose(
                g_ref[pl.ds(i * seq, seq), pl.ds(0, DIM)])
            tyr = ty_ref[pl.ds(i, 1), :].astype(jnp.float32)
            x = xt + pos_t + seg0 + tyr * segd
            mean = jnp.mean(x, axis=0, keepdims=True)
            xc = x - mean
            var = jnp.mean(xc * xc, axis=0, keepdims=True)
            y = xc * lax.rsqrt(var + EPS)
            o_ref[i] = y * gam + bet

    grid = bsz // blk_b
    return pl.pallas_call(
        body,
        grid=(grid,),
        in_specs=[
            pl.BlockSpec((blk_b * seq, 2 * DIM), lambda g: (g, 0)),
            pl.BlockSpec((blk_b, seq), lambda g: (g, 0)),
            pl.BlockSpec((DIM, seq), lambda g: (0, 0)),
            pl.BlockSpec((DIM, 2), lambda g: (0, 0)),
            pl.BlockSpec((DIM, 1), lambda g: (0, 0)),
            pl.BlockSpec((DIM, 1), lambda g: (0, 0)),
        ],
        out_specs=pl.BlockSpec((blk_b, DIM, seq), lambda g: (g, 0, 0)),
        out_shape=jax.ShapeDtypeStruct((bsz, DIM, seq), jnp.float32),
    )


def kernel(input_ids, token_type_ids, token_table, segment_table,
           position_table, ln_gamma, ln_beta):
    bsz, seq = input_ids.shape
    n_tokens = bsz * seq
    vocab, dim = token_table.shape
    assert dim == DIM and n_tokens % (NW * CHUNK) == 0 and seq % 2 == 0

    tok_per_w = n_tokens // NW
    rounds = tok_per_w // CHUNK
    ids = input_ids.reshape(NW, rounds, CHUNK).astype(jnp.int32)

    gath = _build_sc_gather(n_tokens, vocab)(ids, token_table)
    tys = token_type_ids.astype(jnp.int32)

    blk_b = 64
    out_t = _build_tc_ln(bsz, seq, blk_b)(
        gath, tys, position_table.T, segment_table.T,
        ln_gamma[:, None], ln_beta[:, None])
    return out_t.transpose(0, 2, 1)


# speedup vs baseline: 1.2145x; 1.0067x over previous
import functools

import jax
import jax.numpy as jnp
from jax import lax
from jax.experimental import pallas as pl
from jax.experimental.pallas import tpu as pltpu
from jax.experimental.pallas import tpu_sc as plsc

DIM = 64
EPS = 1e-5
NC = 2
NS = 16
NW = NC * NS
CHUNK = 128
NBUF = 5


@functools.lru_cache(maxsize=None)
def _build_sc_gather(n_tokens, vocab):
    tok_per_w = n_tokens // NW
    rounds = tok_per_w // CHUNK
    assert rounds % NBUF == 0

    mesh = plsc.VectorSubcoreMesh(
        core_axis_name="c", subcore_axis_name="s", num_cores=NC, num_subcores=NS
    )

    @functools.partial(
        pl.kernel,
        out_type=jax.ShapeDtypeStruct((n_tokens, 2 * DIM), jnp.float32),
        mesh=mesh,
        scratch_types=[
            pltpu.VMEM((rounds, CHUNK), jnp.int32),
            pltpu.VMEM((NBUF, CHUNK, DIM), jnp.float32),
            [pltpu.SemaphoreType.DMA] * NBUF,
            [pltpu.SemaphoreType.DMA] * NBUF,
        ],
        compiler_params=pltpu.CompilerParams(use_tc_tiling_on_sc=False),
    )
    def sc_gather(ids_hbm, tok_hbm, out_hbm, ids_v, rows_v, gsems, ssems):
        wid = lax.axis_index("s") * NC + lax.axis_index("c")
        base = wid * tok_per_w

        pltpu.sync_copy(ids_hbm.at[wid], ids_v)

        def start_gather(g, b):
            pltpu.async_copy(tok_hbm.at[ids_v.at[g]], rows_v.at[b], gsems[b])

        def wait_gather(g, b):
            pltpu.make_async_copy(
                tok_hbm.at[ids_v.at[g]], rows_v.at[b], gsems[b]).wait()

        def out_slice(g):
            return out_hbm.at[pl.ds(base + g * CHUNK, CHUNK), pl.ds(0, DIM)]

        def start_store(g, b):
            pltpu.async_copy(rows_v.at[b], out_slice(g), ssems[b])

        def wait_store(g, b):
            pltpu.make_async_copy(rows_v.at[b], out_slice(g), ssems[b]).wait()

        for b in range(NBUF):
            start_gather(b, b)

        def ring(r, carry):
            for b in range(NBUF):
                g = NBUF * r + b
                wait_gather(g, b)
                pltpu.sync_copy(rows_v.at[b], out_slice(g))

                @pl.when(g + NBUF < rounds)
                def _():
                    start_gather(g + NBUF, b)
            return carry

        lax.fori_loop(0, rounds // NBUF, ring, 0)

    return sc_gather


@functools.lru_cache(maxsize=None)
def _build_tc_ln(bsz, seq, blk_b):
    def body(g_ref, ty_ref, pos_ref, seg_ref, gam_ref, bet_ref, o_ref):
        pos_t = pos_ref[...]
        seg0 = seg_ref[:, 0:1]
        segd = seg_ref[:, 1:2] - seg0
        gam = gam_ref[...]
        bet = bet_ref[...]
        for i in range(blk_b):
            xt = jnp.transpose(
                g_ref[pl.ds(i * seq, seq), pl.ds(0, DIM)])
            tyr = ty_ref[pl.ds(i, 1), :].astype(jnp.float32)
            x = xt + pos_t + seg0 + tyr * segd
            mean = jnp.mean(x, axis=0, keepdims=True)
            xc = x - mean
            var = jnp.mean(xc * xc, axis=0, keepdims=True)
            y = xc * lax.rsqrt(var + EPS)
            o_ref[i] = y * gam + bet

    grid = bsz // blk_b
    return pl.pallas_call(
        body,
        grid=(grid,),
        in_specs=[
            pl.BlockSpec((blk_b * seq, 2 * DIM), lambda g: (g, 0)),
            pl.BlockSpec((blk_b, seq), lambda g: (g, 0)),
            pl.BlockSpec((DIM, seq), lambda g: (0, 0)),
            pl.BlockSpec((DIM, 2), lambda g: (0, 0)),
            pl.BlockSpec((DIM, 1), lambda g: (0, 0)),
            pl.BlockSpec((DIM, 1), lambda g: (0, 0)),
        ],
        out_specs=pl.BlockSpec((blk_b, DIM, seq), lambda g: (g, 0, 0)),
        out_shape=jax.ShapeDtypeStruct((bsz, DIM, seq), jnp.float32),
    )


def kernel(input_ids, token_type_ids, token_table, segment_table,
           position_table, ln_gamma, ln_beta):
    bsz, seq = input_ids.shape
    n_tokens = bsz * seq
    vocab, dim = token_table.shape
    assert dim == DIM and n_tokens % (NW * CHUNK) == 0 and seq % 2 == 0

    tok_per_w = n_tokens // NW
    rounds = tok_per_w // CHUNK
    ids = input_ids.reshape(NW, rounds, CHUNK).astype(jnp.int32)

    gath = _build_sc_gather(n_tokens, vocab)(ids, token_table)
    tys = token_type_ids.astype(jnp.int32)

    blk_b = 128
    out_t = _build_tc_ln(bsz, seq, blk_b)(
        gath, tys, position_table.T, segment_table.T,
        ln_gamma[:, None], ln_beta[:, None])
    return out_t.transpose(0, 2, 1)
